# Initial kernel scaffold; baseline (speedup 1.0000x reference)
import jax
import jax.numpy as jnp
from jax.experimental import pallas as pl


def kernel(x, W, b):
    raise NotImplementedError("write your pallas kernel here")


# speedup vs baseline: 128.1488x; 128.1488x over previous
import functools

import jax
import jax.numpy as jnp
from jax import lax
from jax.experimental import pallas as pl
from jax.experimental.pallas import tpu as pltpu

_M = 16
_K = 3
_D = 512
_EXCL = 4
_TILE_R = 256


def _better(av, af, bv, bf):
    return (av > bv) | ((av == bv) & (af < bf))


def _mp_kernel(ts_ref, x_ref, w_ref, b_ref, out_ref,
               wz_ref, vals_ref, flats_ref, cols_ref, *, n, s_len, n_rt):
    r = pl.program_id(1)

    @pl.when(r == 0)
    def _init():
        w = jnp.stack([ts_ref[0, 0, pl.ds(k, s_len)] for k in range(_M)], axis=0)
        mu = jnp.mean(w, axis=0, keepdims=True)
        sd = jnp.sqrt(jnp.mean((w - mu) ** 2, axis=0, keepdims=True)) + 1e-8
        wz_ref[...] = (w - mu) / sd
        for i in range(_K):
            vals_ref[i] = -jnp.inf
            flats_ref[i] = jnp.int32(2**31 - 1)
            cols_ref[i] = jnp.int32(0)

    r0 = r * _TILE_R
    lhs = wz_ref[:, pl.ds(r0, _TILE_R)]
    dot = lax.dot_general(lhs, wz_ref[...], (((0,), (0,)), ((), ())),
                          preferred_element_type=jnp.float32)
    rows = r0 + lax.broadcasted_iota(jnp.int32, dot.shape, 0)
    colm = lax.broadcasted_iota(jnp.int32, dot.shape, 1)
    valid = (rows < n) & (colm < n) & (jnp.abs(rows - colm) > _EXCL)
    score = jnp.where(valid, dot, -jnp.inf)
    flat = rows * n + colm

    v = [vals_ref[i] for i in range(_K)]
    f = [flats_ref[i] for i in range(_K)]
    c = [cols_ref[i] for i in range(_K)]
    big = jnp.int32(2**31 - 1)
    for _ in range(_K):
        pv = jnp.max(score)
        hit = score == pv
        pf = jnp.min(jnp.where(hit, flat, big))
        pc = jnp.min(jnp.where(hit & (flat == pf), colm, big))
        b0 = _better(pv, pf, v[0], f[0])
        b1 = _better(pv, pf, v[1], f[1])
        b2 = _better(pv, pf, v[2], f[2])
        nv0 = jnp.where(b0, pv, v[0])
        nf0 = jnp.where(b0, pf, f[0])
        nc0 = jnp.where(b0, pc, c[0])
        nv1 = jnp.where(b0, v[0], jnp.where(b1, pv, v[1]))
        nf1 = jnp.where(b0, f[0], jnp.where(b1, pf, f[1]))
        nc1 = jnp.where(b0, c[0], jnp.where(b1, pc, c[1]))
        nv2 = jnp.where(b1, v[1], jnp.where(b2, pv, v[2]))
        nf2 = jnp.where(b1, f[1], jnp.where(b2, pf, f[2]))
        nc2 = jnp.where(b1, c[1], jnp.where(b2, pc, c[2]))
        v = [nv0, nv1, nv2]
        f = [nf0, nf1, nf2]
        c = [nc0, nc1, nc2]
        score = jnp.where(flat == pf, -jnp.inf, score)

    for i in range(_K):
        vals_ref[i] = v[i]
        flats_ref[i] = f[i]
        cols_ref[i] = c[i]

    @pl.when(r == n_rt - 1)
    def _emit():
        for kk in range(_K):
            st = jnp.clip(c[kk] - _M // 2, 0, s_len - _M)
            patch = x_ref[0, pl.ds(st, _M), :]
            ok = lax.dot_general(patch, w_ref[...], (((0,), (1,)), ((), ())),
                                 preferred_element_type=jnp.float32)
            out_ref[0, kk, :, :] = ok + b_ref[0, :][None, :]


def kernel(x, W, b):
    B, S, C = x.shape
    n = S - _M + 1
    n_rt = S // _TILE_R
    ts_pad = jnp.pad(x[:, :, 0], ((0, 0), (0, 128))).reshape(B, 1, S + 128)
    bias2d = b.reshape(1, _D)
    out = pl.pallas_call(
        functools.partial(_mp_kernel, n=n, s_len=S, n_rt=n_rt),
        grid=(B, n_rt),
        in_specs=[
            pl.BlockSpec((1, 1, S + 128), lambda bb, r: (bb, 0, 0)),
            pl.BlockSpec((1, S, C), lambda bb, r: (bb, 0, 0)),
            pl.BlockSpec((_D, _M), lambda bb, r: (0, 0)),
            pl.BlockSpec((1, _D), lambda bb, r: (0, 0)),
        ],
        out_specs=pl.BlockSpec((1, _K, C, _D), lambda bb, r: (bb, 0, 0, 0)),
        out_shape=jax.ShapeDtypeStruct((B, _K, C, _D), jnp.float32),
        scratch_shapes=[
            pltpu.VMEM((_M, S), jnp.float32),
            pltpu.SMEM((_K,), jnp.float32),
            pltpu.SMEM((_K,), jnp.int32),
            pltpu.SMEM((_K,), jnp.int32),
        ],
        compiler_params=pltpu.CompilerParams(
            dimension_semantics=("arbitrary", "arbitrary")),
    )(ts_pad, x, W, bias2d)
    return jnp.transpose(out, (0, 2, 1, 3))


# speedup vs baseline: 1397.9335x; 10.9087x over previous
import functools

import jax
import jax.numpy as jnp
import numpy as np
from jax import lax
from jax.experimental import pallas as pl
from jax.experimental.pallas import tpu as pltpu

_M = 16
_K = 3
_D = 512
_EXCL = 4
_T = 256
_NEG = np.float32(-np.inf)


def _better(av, af, bv, bf):
    return (av > bv) | ((av == bv) & (af < bf))


def _mp_kernel(ts_ref, x_ref, w_ref, b_ref, out_ref,
               wz3_ref, masks_ref, *, n, s_len):
    nt = s_len // _T
    lastc0 = (nt - 1) * _T

    w = jnp.stack([ts_ref[0, 0, pl.ds(k, s_len)] for k in range(_M)], axis=0)
    mu = jnp.mean(w, axis=0, keepdims=True)
    sd = jnp.sqrt(jnp.mean((w - mu) ** 2, axis=0, keepdims=True)) + 1e-8
    wz = (w - mu) / sd
    for c in range(nt):
        wz3_ref[c] = wz[:, c * _T:(c + 1) * _T]

    r_io = lax.broadcasted_iota(jnp.int32, (_T, _T), 0)
    c_io = lax.broadcasted_iota(jnp.int32, (_T, _T), 1)
    zero = jnp.zeros((_T, _T), jnp.float32)
    diag_m = jnp.where(c_io - r_io > _EXCL, 0.0, _NEG)
    sup_m = jnp.where(c_io + _T - r_io > _EXCL, 0.0, _NEG)
    last_m = jnp.where(c_io < n - lastc0, 0.0, _NEG)
    masks_ref[0] = zero
    masks_ref[1] = diag_m
    masks_ref[2] = sup_m
    masks_ref[3] = last_m
    masks_ref[4] = diag_m + last_m
    masks_ref[5] = sup_m + last_m

    def tile_class(rt, ct):
        if ct == rt:
            return 4 if ct == nt - 1 else 1
        if ct == rt + 1:
            return 5 if ct == nt - 1 else 2
        if ct == nt - 1:
            return 3
        return 0
    class_mask = [zero, diag_m, sup_m, last_m, diag_m + last_m, sup_m + last_m]

    ma = np.float32(-np.inf)
    mb = np.float32(-np.inf)
    rta = np.int32(0); cta = np.int32(0); mda = np.int32(0)
    rtb = np.int32(0); ctb = np.int32(0); mdb = np.int32(0)
    for rt in range(nt):
        for ct in range(rt, nt):
            md = tile_class(rt, ct)
            d = lax.dot_general(wz3_ref[rt], wz3_ref[ct],
                                (((0,), (0,)), ((), ())),
                                preferred_element_type=jnp.float32)
            if md == 0:
                pv = jnp.max(d)
            else:
                pv = jnp.max(d + class_mask[md])
            ta = pv > ma
            tb = pv > mb
            mb = jnp.where(ta, ma, jnp.where(tb, pv, mb))
            rtb = jnp.where(ta, rta, jnp.where(tb, rt, rtb))
            ctb = jnp.where(ta, cta, jnp.where(tb, ct, ctb))
            mdb = jnp.where(ta, mda, jnp.where(tb, md, mdb))
            ma = jnp.where(ta, pv, ma)
            rta = jnp.where(ta, rt, rta)
            cta = jnp.where(ta, ct, cta)
            mda = jnp.where(ta, md, mda)

    big = np.int32(2**31 - 1)

    def tile_score(rt, ct, md):
        d = lax.dot_general(wz3_ref[rt], wz3_ref[ct],
                            (((0,), (0,)), ((), ())),
                            preferred_element_type=jnp.float32)
        score = d + masks_ref[md]
        rows = rt * _T + r_io
        cols = ct * _T + c_io
        flat2 = jnp.left_shift(rows, 11) | cols
        return score, flat2

    score_a, flat_a = tile_score(rta, cta, mda)
    pv1 = jnp.max(score_a)
    pf1 = jnp.min(jnp.where(score_a == pv1, flat_a, big))
    score_a2 = jnp.where(flat_a == pf1, _NEG, score_a)
    pv2 = jnp.max(score_a2)
    pf2 = jnp.min(jnp.where(score_a2 == pv2, flat_a, big))

    score_b, flat_b = tile_score(rtb, ctb, mdb)
    pv3 = jnp.max(score_b)
    pf3 = jnp.min(jnp.where(score_b == pv3, flat_b, big))

    use2 = _better(pv2, pf2, pv3, pf3)
    f0 = pf1
    f1 = jnp.where(use2, pf2, pf3)

    i1 = jnp.right_shift(f0, 11)
    j1 = jnp.bitwise_and(f0, 2047)
    j2 = jnp.bitwise_and(f1, 2047)

    for kk, cc in enumerate((j1, i1, j2)):
        st = jnp.clip(cc - _M // 2, 0, s_len - _M)
        patch = x_ref[0, pl.ds(st, _M), :]
        ok = lax.dot_general(patch, w_ref[...], (((0,), (1,)), ((), ())),
                             preferred_element_type=jnp.float32)
        out_ref[0, kk, :, :] = ok + b_ref[0, :][None, :]


def kernel(x, W, b):
    B, S, C = x.shape
    n = S - _M + 1
    nt = S // _T
    ts_pad = jnp.pad(x[:, :, 0], ((0, 0), (0, 128))).reshape(B, 1, S + 128)
    bias2d = b.reshape(1, _D)
    out = pl.pallas_call(
        functools.partial(_mp_kernel, n=n, s_len=S),
        grid=(B,),
        in_specs=[
            pl.BlockSpec((1, 1, S + 128), lambda bb: (bb, 0, 0)),
            pl.BlockSpec((1, S, C), lambda bb: (bb, 0, 0)),
            pl.BlockSpec((_D, _M), lambda bb: (0, 0)),
            pl.BlockSpec((1, _D), lambda bb: (0, 0)),
        ],
        out_specs=pl.BlockSpec((1, _K, C, _D), lambda bb: (bb, 0, 0, 0)),
        out_shape=jax.ShapeDtypeStruct((B, _K, C, _D), jnp.float32),
        scratch_shapes=[
            pltpu.VMEM((nt, _M, _T), jnp.float32),
            pltpu.VMEM((6, _T, _T), jnp.float32),
        ],
        compiler_params=pltpu.CompilerParams(
            dimension_semantics=("arbitrary",)),
    )(ts_pad, x, W, bias2d)
    return jnp.transpose(out, (0, 2, 1, 3))


# speedup vs baseline: 1398.3397x; 1.0003x over previous
import functools

import jax
import jax.numpy as jnp
import numpy as np
from jax import lax
from jax.experimental import pallas as pl
from jax.experimental.pallas import tpu as pltpu

_M = 16
_K = 3
_D = 512
_EXCL = 4
_T = 256
_NEG = np.float32(-np.inf)


def _better(av, af, bv, bf):
    return (av > bv) | ((av == bv) & (af < bf))


def _mp_kernel(ts_ref, x_ref, w_ref, b_ref, out_ref,
               wz3_ref, masks_ref, *, n, s_len):
    nt = s_len // _T
    lastc0 = (nt - 1) * _T

    w = jnp.stack([ts_ref[0, 0, pl.ds(k, s_len)] for k in range(_M)], axis=0)
    mu = jnp.mean(w, axis=0, keepdims=True)
    sd = jnp.sqrt(jnp.mean((w - mu) ** 2, axis=0, keepdims=True)) + 1e-8
    wz = (w - mu) / sd
    for c in range(nt):
        wz3_ref[c] = wz[:, c * _T:(c + 1) * _T]

    r_io = lax.broadcasted_iota(jnp.int32, (_T, _T), 0)
    c_io = lax.broadcasted_iota(jnp.int32, (_T, _T), 1)
    zero = jnp.zeros((_T, _T), jnp.float32)
    diag_m = jnp.where(c_io - r_io > _EXCL, 0.0, _NEG)
    sup_m = jnp.where(c_io + _T - r_io > _EXCL, 0.0, _NEG)
    last_m = jnp.where(c_io < n - lastc0, 0.0, _NEG)
    masks_ref[0] = zero
    masks_ref[1] = diag_m
    masks_ref[2] = sup_m
    masks_ref[3] = last_m
    masks_ref[4] = diag_m + last_m
    masks_ref[5] = sup_m + last_m

    def tile_class(rt, ct):
        if ct == rt:
            return 4 if ct == nt - 1 else 1
        if ct == rt + 1:
            return 5 if ct == nt - 1 else 2
        if ct == nt - 1:
            return 3
        return 0
    class_mask = [zero, diag_m, sup_m, last_m, diag_m + last_m, sup_m + last_m]

    ma = np.float32(-np.inf)
    mb = np.float32(-np.inf)
    rta = np.int32(0); cta = np.int32(0); mda = np.int32(0)
    rtb = np.int32(0); ctb = np.int32(0); mdb = np.int32(0)
    for rt in range(nt):
        for ct in range(rt, nt):
            md = tile_class(rt, ct)
            d = lax.dot_general(wz3_ref[rt], wz3_ref[ct],
                                (((0,), (0,)), ((), ())),
                                preferred_element_type=jnp.float32)
            if md == 0:
                pv = jnp.max(d)
            else:
                pv = jnp.max(d + class_mask[md])
            ta = pv > ma
            tb = pv > mb
            mb = jnp.where(ta, ma, jnp.where(tb, pv, mb))
            rtb = jnp.where(ta, rta, jnp.where(tb, rt, rtb))
            ctb = jnp.where(ta, cta, jnp.where(tb, ct, ctb))
            mdb = jnp.where(ta, mda, jnp.where(tb, md, mdb))
            ma = jnp.where(ta, pv, ma)
            rta = jnp.where(ta, rt, rta)
            cta = jnp.where(ta, ct, cta)
            mda = jnp.where(ta, md, mda)

    big = np.int32(2**31 - 1)

    def tile_score(rt, ct, md):
        d = lax.dot_general(wz3_ref[rt], wz3_ref[ct],
                            (((0,), (0,)), ((), ())),
                            preferred_element_type=jnp.float32)
        score = d + masks_ref[md]
        rows = rt * _T + r_io
        cols = ct * _T + c_io
        flat2 = jnp.left_shift(rows, 11) | cols
        return score, flat2

    score_a, flat_a = tile_score(rta, cta, mda)
    pv1 = jnp.max(score_a)
    pf1 = jnp.min(jnp.where(score_a == pv1, flat_a, big))
    score_a2 = jnp.where(flat_a == pf1, _NEG, score_a)
    pv2 = jnp.max(score_a2)
    pf2 = jnp.min(jnp.where(score_a2 == pv2, flat_a, big))

    score_b, flat_b = tile_score(rtb, ctb, mdb)
    pv3 = jnp.max(score_b)
    pf3 = jnp.min(jnp.where(score_b == pv3, flat_b, big))

    use2 = _better(pv2, pf2, pv3, pf3)
    f0 = pf1
    f1 = jnp.where(use2, pf2, pf3)

    i1 = jnp.right_shift(f0, 11)
    j1 = jnp.bitwise_and(f0, 2047)
    j2 = jnp.bitwise_and(f1, 2047)

    for kk, cc in enumerate((j1, i1, j2)):
        st = jnp.clip(cc - _M // 2, 0, s_len - _M)
        patch = x_ref[0, pl.ds(st, _M), :]
        ok = lax.dot_general(patch, w_ref[...], (((0,), (1,)), ((), ())),
                             preferred_element_type=jnp.float32)
        out_ref[0, kk, :, :] = ok + b_ref[0, :][None, :]


def kernel(x, W, b):
    B, S, C = x.shape
    n = S - _M + 1
    nt = S // _T
    ts_pad = jnp.pad(x[:, :, 0], ((0, 0), (0, 128))).reshape(B, 1, S + 128)
    bias2d = b.reshape(1, _D)
    out = pl.pallas_call(
        functools.partial(_mp_kernel, n=n, s_len=S),
        grid=(B,),
        in_specs=[
            pl.BlockSpec((1, 1, S + 128), lambda bb: (bb, 0, 0)),
            pl.BlockSpec((1, S, C), lambda bb: (bb, 0, 0)),
            pl.BlockSpec((_D, _M), lambda bb: (0, 0)),
            pl.BlockSpec((1, _D), lambda bb: (0, 0)),
        ],
        out_specs=pl.BlockSpec((1, _K, C, _D), lambda bb: (bb, 0, 0, 0)),
        out_shape=jax.ShapeDtypeStruct((B, _K, C, _D), jnp.float32),
        scratch_shapes=[
            pltpu.VMEM((nt, _M, _T), jnp.float32),
            pltpu.VMEM((6, _T, _T), jnp.float32),
        ],
        compiler_params=pltpu.CompilerParams(
            dimension_semantics=("parallel",)),
    )(ts_pad, x, W, bias2d)
    return jnp.transpose(out, (0, 2, 1, 3))
